# baseline (speedup 1.0000x reference)
import functools

import jax
import jax.numpy as jnp
from jax import lax
from jax.experimental import pallas as pl
from jax.experimental.pallas import tpu as pltpu
from jax.experimental.pallas import tpu_sc as plsc

NUM_CORES = 2
NUM_SUBCORES = 16
NUM_WORKERS = NUM_CORES * NUM_SUBCORES
BATCH = 16384
EMBED_DIM = 6
LANES = 16
B_PER_W = BATCH // NUM_WORKERS
W_PER_W = B_PER_W * EMBED_DIM
CHUNK = 128
N_STREAMS = W_PER_W // CHUNK
N_GROUPS = B_PER_W // LANES

_mesh = plsc.VectorSubcoreMesh(
    core_axis_name="c", subcore_axis_name="s",
    num_cores=NUM_CORES, num_subcores=NUM_SUBCORES,
)


@functools.partial(
    pl.kernel,
    out_type=jax.ShapeDtypeStruct((BATCH * EMBED_DIM,), jnp.float32),
    mesh=_mesh,
    compiler_params=pltpu.CompilerParams(
        use_tc_tiling_on_sc=False, needs_layout_passes=False),
    scratch_types=[
        pltpu.VMEM((B_PER_W,), jnp.int32),
        pltpu.VMEM((W_PER_W,), jnp.int32),
        pltpu.VMEM((W_PER_W,), jnp.float32),
        pltpu.SemaphoreType.DMA,
    ],
)
def _sc_gather(idx_hbm, tab_hbm, out_hbm, idx_v, addr_v, rows_v, sem):
    wid = lax.axis_index("s") * NUM_CORES + lax.axis_index("c")
    base = wid * B_PER_W
    pltpu.sync_copy(idx_hbm.at[pl.ds(base, B_PER_W)], idx_v)
    lane6 = lax.iota(jnp.int32, LANES) * EMBED_DIM
    for g in range(N_GROUPS):
        v6 = idx_v[pl.ds(g * LANES, LANES)] * EMBED_DIM
        pos = lane6 + (g * LANES * EMBED_DIM)
        for k in range(EMBED_DIM):
            plsc.store_scatter(addr_v, [pos + k], v6 + k)
    copies = []
    for s in range(N_STREAMS):
        copies.append(
            pltpu.async_copy(
                tab_hbm.at[addr_v.at[pl.ds(s * CHUNK, CHUNK)]],
                rows_v.at[pl.ds(s * CHUNK, CHUNK)],
                sem,
            )
        )
    for c in copies:
        c.wait()
    pltpu.sync_copy(rows_v, out_hbm.at[pl.ds(base * EMBED_DIM, W_PER_W)])


def kernel(indices, table):
    idx = indices.astype(jnp.int32)
    flat = table.reshape(-1)
    out = _sc_gather(idx, flat)
    return out.reshape(BATCH, EMBED_DIM)
